# baseline (speedup 1.0000x reference)
import functools

import jax
import jax.numpy as jnp
import numpy as np
from jax import lax
from jax.experimental import pallas as pl
from jax.experimental.pallas import tpu as pltpu
from jax.experimental.pallas import tpu_sc as plsc

_C0 = np.float32(0.044715)
_C1 = np.float32(0.7978845608028654)


def _gelu(v):
    s3 = jnp.minimum(_C0 * ((v * v) * v), np.float32(3.0e38))
    t = jnp.tanh(_C1 * (v + s3))
    return (np.float32(0.5) * v) * (t + np.float32(1.0))


def _mlp_body(x_ref, w0_ref, b0_ref, w1_ref, b1_ref, wat_ref, alpha_ref):
    xb = x_ref[...].astype(jnp.bfloat16)
    h = _gelu(jnp.dot(xb, w0_ref[...], preferred_element_type=jnp.float32)
              + b0_ref[...])
    h2 = _gelu(jnp.dot(h.astype(jnp.bfloat16), w1_ref[...],
                       preferred_element_type=jnp.float32) + b1_ref[...])
    alpha_ref[...] = lax.dot_general(wat_ref[...], h2.astype(jnp.bfloat16),
                                     (((1,), (1,)), ((), ())),
                                     preferred_element_type=jnp.float32)


def _mlp_scores(x2d, W0, b0, W1, b1, Wa, block_m):
    m, d = x2d.shape
    d0 = W0.shape[1]
    d1 = W1.shape[1]
    grid = (m // block_m,)
    return pl.pallas_call(
        _mlp_body,
        grid=grid,
        in_specs=[
            pl.BlockSpec((block_m, d), lambda i: (i, 0)),
            pl.BlockSpec((d, d0), lambda i: (0, 0)),
            pl.BlockSpec((1, d0), lambda i: (0, 0)),
            pl.BlockSpec((d0, d1), lambda i: (0, 0)),
            pl.BlockSpec((1, d1), lambda i: (0, 0)),
            pl.BlockSpec((1, d1), lambda i: (0, 0)),
        ],
        out_specs=pl.BlockSpec((1, block_m), lambda i: (0, i)),
        out_shape=jax.ShapeDtypeStruct((1, m), jnp.float32),
    )(x2d, W0.astype(jnp.bfloat16), b0.reshape(1, d0),
      W1.astype(jnp.bfloat16), b1.reshape(1, d1),
      Wa.reshape(1, d1).astype(jnp.bfloat16))


def _select_body(alpha_ref, alpha_t_ref, sel_ref, gidx_ref, *, batch, seq, topk,
                 chunk):
    i_row = lax.broadcasted_iota(jnp.int32, (1, seq), 1)
    for b in range(batch):
        v = alpha_ref[b:b + 1, :]
        m = jnp.max(v)
        e = jnp.exp(v - m)
        s = jnp.sum(e)
        p_row = e / s
        rank = jnp.zeros((1, seq), jnp.int32)
        for c in range(seq // chunk):
            a_col = alpha_t_ref[c * chunk:(c + 1) * chunk, b:b + 1]
            p_col = jnp.exp(a_col - m) / s
            j_col = (lax.broadcasted_iota(jnp.int32, (chunk, 1), 0)
                     + c * chunk)
            beats = (p_col > p_row) | ((p_col == p_row) & (j_col < i_row))
            rank = rank + jnp.sum(beats.astype(jnp.int32), axis=0,
                                  keepdims=True)
        r_col = lax.broadcasted_iota(jnp.int32, (topk, 1), 0)
        hit = rank == r_col
        sel_col = jnp.sum(jnp.where(hit, i_row, 0), axis=1, keepdims=True)
        sel_ref[:, b:b + 1] = sel_col
        gidx_ref[:, b:b + 1] = sel_col + b * seq


def _select(alpha, alpha_t, topk):
    batch, seq = alpha.shape
    body = functools.partial(_select_body, batch=batch, seq=seq, topk=topk,
                             chunk=256)
    return pl.pallas_call(
        body,
        out_shape=(
            jax.ShapeDtypeStruct((topk, batch), jnp.int32),
            jax.ShapeDtypeStruct((topk, batch), jnp.int32),
        ),
    )(alpha, alpha_t)


def _sc_gather(x2d, gidx, num_rows):
    d = x2d.shape[1]
    info = plsc.get_sparse_core_info()
    nw = info.num_cores * info.num_subcores
    rows_per_w = num_rows // nw
    mesh = plsc.VectorSubcoreMesh(core_axis_name="c", subcore_axis_name="s")

    @functools.partial(
        pl.kernel,
        mesh=mesh,
        out_type=jax.ShapeDtypeStruct((num_rows, d), jnp.float32),
        scratch_types=[
            pltpu.VMEM((rows_per_w,), jnp.int32),
            pltpu.VMEM((rows_per_w, d), jnp.float32),
            pltpu.SemaphoreType.DMA,
        ],
    )
    def gather_kernel(table_hbm, idx_hbm, out_hbm, idx_v, rows_v, sem):
        wid = lax.axis_index("s") * info.num_cores + lax.axis_index("c")
        base = wid * rows_per_w
        pltpu.sync_copy(idx_hbm.at[pl.ds(base, rows_per_w)], idx_v)
        pltpu.async_copy(table_hbm.at[idx_v], rows_v, sem).wait()
        pltpu.sync_copy(rows_v, out_hbm.at[pl.ds(base, rows_per_w)])

    return gather_kernel(x2d, gidx)


def kernel(x, W0, b0, W1, b1, Wa, ba):
    topk = 256
    batch, seq, d = x.shape
    x2d = x.reshape(batch * seq, d)
    alpha = _mlp_scores(x2d, W0, b0, W1, b1, Wa, block_m=512)
    alpha = alpha.reshape(batch, seq)
    sel_t, gidx_t = _select(alpha, alpha.T, topk)
    sel_indices = sel_t.T
    gidx = gidx_t.T.reshape(batch * topk)
    cand2d = _sc_gather(x2d, gidx, batch * topk)
    return sel_indices, cand2d.reshape(batch, topk, d)


# speedup vs baseline: 1.0582x; 1.0582x over previous
import functools

import jax
import jax.numpy as jnp
import numpy as np
from jax import lax
from jax.experimental import pallas as pl
from jax.experimental.pallas import tpu as pltpu
from jax.experimental.pallas import tpu_sc as plsc

_C0 = np.float32(0.044715)
_C1 = np.float32(0.7978845608028654)


def _gelu(v):
    s3 = jnp.minimum(_C0 * ((v * v) * v), np.float32(3.0e38))
    t = jnp.tanh(_C1 * (v + s3))
    return (np.float32(0.5) * v) * (t + np.float32(1.0))


def _mlp_body(x_ref, w0_ref, b0_ref, w1_ref, b1_ref, wat_ref, alpha_ref):
    h = _gelu(jnp.dot(x_ref[...], w0_ref[...]) + b0_ref[...])
    h2 = _gelu(jnp.dot(h, w1_ref[...]) + b1_ref[...])
    alpha_ref[...] = lax.dot_general(wat_ref[...], h2,
                                     (((1,), (1,)), ((), ())))


def _mlp_scores(x2d, W0, b0, W1, b1, Wa, block_m):
    m, d = x2d.shape
    d0 = W0.shape[1]
    d1 = W1.shape[1]
    grid = (m // block_m,)
    return pl.pallas_call(
        _mlp_body,
        grid=grid,
        in_specs=[
            pl.BlockSpec((block_m, d), lambda i: (i, 0)),
            pl.BlockSpec((d, d0), lambda i: (0, 0)),
            pl.BlockSpec((1, d0), lambda i: (0, 0)),
            pl.BlockSpec((d0, d1), lambda i: (0, 0)),
            pl.BlockSpec((1, d1), lambda i: (0, 0)),
            pl.BlockSpec((1, d1), lambda i: (0, 0)),
        ],
        out_specs=pl.BlockSpec((1, block_m), lambda i: (0, i)),
        out_shape=jax.ShapeDtypeStruct((1, m), jnp.float32),
    )(x2d, W0, b0.reshape(1, d0), W1, b1.reshape(1, d1), Wa.reshape(1, d1))


def _select_body(alpha_ref, alpha_t_ref, sel_ref, gidx_ref, *, batch, seq, topk,
                 chunk):
    i_row = lax.broadcasted_iota(jnp.int32, (1, seq), 1)
    for b in range(batch):
        v = alpha_ref[b:b + 1, :]
        m = jnp.max(v)
        e = jnp.exp(v - m)
        s = jnp.sum(e)
        p_row = e / s
        rank = jnp.zeros((1, seq), jnp.int32)
        for c in range(seq // chunk):
            a_col = alpha_t_ref[c * chunk:(c + 1) * chunk, b:b + 1]
            p_col = jnp.exp(a_col - m) / s
            j_col = (lax.broadcasted_iota(jnp.int32, (chunk, 1), 0)
                     + c * chunk)
            beats = (p_col > p_row) | ((p_col == p_row) & (j_col < i_row))
            rank = rank + jnp.sum(beats.astype(jnp.int32), axis=0,
                                  keepdims=True)
        r_col = lax.broadcasted_iota(jnp.int32, (topk, 1), 0)
        hit = rank == r_col
        sel_col = jnp.sum(jnp.where(hit, i_row, 0), axis=1, keepdims=True)
        sel_ref[:, b:b + 1] = sel_col
        gidx_ref[:, b:b + 1] = sel_col + b * seq


def _select(alpha, alpha_t, topk):
    batch, seq = alpha.shape
    body = functools.partial(_select_body, batch=batch, seq=seq, topk=topk,
                             chunk=256)
    return pl.pallas_call(
        body,
        out_shape=(
            jax.ShapeDtypeStruct((topk, batch), jnp.int32),
            jax.ShapeDtypeStruct((topk, batch), jnp.int32),
        ),
    )(alpha, alpha_t)


def _sc_gather(x2d, gidx, num_rows):
    d = x2d.shape[1]
    info = plsc.get_sparse_core_info()
    nw = info.num_cores * info.num_subcores
    rows_per_w = num_rows // nw
    mesh = plsc.VectorSubcoreMesh(core_axis_name="c", subcore_axis_name="s")

    @functools.partial(
        pl.kernel,
        mesh=mesh,
        out_type=jax.ShapeDtypeStruct((num_rows, d), jnp.float32),
        scratch_types=[
            pltpu.VMEM((rows_per_w,), jnp.int32),
            pltpu.VMEM((rows_per_w, d), jnp.float32),
            pltpu.SemaphoreType.DMA,
        ],
    )
    def gather_kernel(table_hbm, idx_hbm, out_hbm, idx_v, rows_v, sem):
        wid = lax.axis_index("s") * info.num_cores + lax.axis_index("c")
        base = wid * rows_per_w
        pltpu.sync_copy(idx_hbm.at[pl.ds(base, rows_per_w)], idx_v)
        pltpu.async_copy(table_hbm.at[idx_v], rows_v, sem).wait()
        pltpu.sync_copy(rows_v, out_hbm.at[pl.ds(base, rows_per_w)])

    return gather_kernel(x2d, gidx)


def kernel(x, W0, b0, W1, b1, Wa, ba):
    topk = 256
    batch, seq, d = x.shape
    x2d = x.reshape(batch * seq, d)
    alpha = _mlp_scores(x2d, W0, b0, W1, b1, Wa, block_m=1024)
    alpha = alpha.reshape(batch, seq)
    sel_t, gidx_t = _select(alpha, alpha.T, topk)
    sel_indices = sel_t.T
    gidx = gidx_t.T.reshape(batch * topk)
    cand2d = _sc_gather(x2d, gidx, batch * topk)
    return sel_indices, cand2d.reshape(batch, topk, d)


# speedup vs baseline: 1.0991x; 1.0387x over previous
import functools

import jax
import jax.numpy as jnp
import numpy as np
from jax import lax
from jax.experimental import pallas as pl
from jax.experimental.pallas import tpu as pltpu
from jax.experimental.pallas import tpu_sc as plsc

_C0 = np.float32(0.044715)
_C1 = np.float32(0.7978845608028654)


def _gelu(v):
    s3 = jnp.minimum(_C0 * ((v * v) * v), np.float32(3.0e38))
    t = jnp.tanh(_C1 * (v + s3))
    return (np.float32(0.5) * v) * (t + np.float32(1.0))


def _mlp_body(x_ref, w0_ref, b0_ref, w1_ref, b1_ref, wat_ref, alpha_ref):
    h = _gelu(jnp.dot(x_ref[...], w0_ref[...]) + b0_ref[...])
    h2 = _gelu(jnp.dot(h, w1_ref[...]) + b1_ref[...])
    alpha_ref[...] = lax.dot_general(wat_ref[...], h2,
                                     (((1,), (1,)), ((), ())))


def _mlp_scores(x2d, W0, b0, W1, b1, Wa, block_m):
    m, d = x2d.shape
    d0 = W0.shape[1]
    d1 = W1.shape[1]
    grid = (m // block_m,)
    return pl.pallas_call(
        _mlp_body,
        grid=grid,
        in_specs=[
            pl.BlockSpec((block_m, d), lambda i: (i, 0)),
            pl.BlockSpec((d, d0), lambda i: (0, 0)),
            pl.BlockSpec((1, d0), lambda i: (0, 0)),
            pl.BlockSpec((d0, d1), lambda i: (0, 0)),
            pl.BlockSpec((1, d1), lambda i: (0, 0)),
            pl.BlockSpec((1, d1), lambda i: (0, 0)),
        ],
        out_specs=pl.BlockSpec((1, block_m), lambda i: (0, i)),
        out_shape=jax.ShapeDtypeStruct((1, m), jnp.float32),
    )(x2d, W0, b0.reshape(1, d0), W1, b1.reshape(1, d1), Wa.reshape(1, d1))


def _select_body(alpha_ref, sel_ref, gidx_ref, *, batch, seq, topk, chunk):
    j_row = lax.broadcasted_iota(jnp.int32, (1, seq), 1)
    r_row = lax.broadcasted_iota(jnp.int32, (1, topk), 1)
    for b in range(batch):
        v = alpha_ref[0:1, b * seq:(b + 1) * seq]
        m = jnp.max(v)
        e = jnp.exp(v - m)
        s = jnp.sum(e)
        p_row = e / s
        sel_acc = jnp.zeros((1, topk), jnp.int32)
        for c in range(seq // chunk):
            a_col = jnp.reshape(
                alpha_ref[0:1, b * seq + c * chunk:b * seq + (c + 1) * chunk],
                (chunk, 1))
            p_col = jnp.exp(a_col - m) / s
            i_col = (lax.broadcasted_iota(jnp.int32, (chunk, 1), 0)
                     + c * chunk)
            beats = (p_row > p_col) | ((p_row == p_col) & (j_row < i_col))
            rank_c = jnp.sum(beats.astype(jnp.int32), axis=1, keepdims=True)
            hit = rank_c == r_row
            sel_acc = sel_acc + jnp.sum(jnp.where(hit, i_col, 0), axis=0,
                                        keepdims=True)
        sel_ref[b:b + 1, :] = sel_acc
        gidx_ref[0:1, b * topk:(b + 1) * topk] = sel_acc + b * seq


def _select(alpha, batch, seq, topk):
    body = functools.partial(_select_body, batch=batch, seq=seq, topk=topk,
                             chunk=256)
    return pl.pallas_call(
        body,
        out_shape=(
            jax.ShapeDtypeStruct((batch, topk), jnp.int32),
            jax.ShapeDtypeStruct((1, batch * topk), jnp.int32),
        ),
    )(alpha)


def _sc_gather(x2d, gidx, num_rows):
    d = x2d.shape[1]
    info = plsc.get_sparse_core_info()
    nw = info.num_cores * info.num_subcores
    rows_per_w = num_rows // nw
    mesh = plsc.VectorSubcoreMesh(core_axis_name="c", subcore_axis_name="s")

    @functools.partial(
        pl.kernel,
        mesh=mesh,
        out_type=jax.ShapeDtypeStruct((num_rows, d), jnp.float32),
        scratch_types=[
            pltpu.VMEM((rows_per_w,), jnp.int32),
            pltpu.VMEM((rows_per_w, d), jnp.float32),
            pltpu.SemaphoreType.DMA,
        ],
    )
    def gather_kernel(table_hbm, idx_hbm, out_hbm, idx_v, rows_v, sem):
        wid = lax.axis_index("s") * info.num_cores + lax.axis_index("c")
        base = wid * rows_per_w
        pltpu.sync_copy(idx_hbm.at[pl.ds(base, rows_per_w)], idx_v)
        pltpu.async_copy(table_hbm.at[idx_v], rows_v, sem).wait()
        pltpu.sync_copy(rows_v, out_hbm.at[pl.ds(base, rows_per_w)])

    return gather_kernel(x2d, gidx)


def kernel(x, W0, b0, W1, b1, Wa, ba):
    topk = 256
    batch, seq, d = x.shape
    x2d = x.reshape(batch * seq, d)
    alpha = _mlp_scores(x2d, W0, b0, W1, b1, Wa, block_m=1024)
    sel_indices, gidx_row = _select(alpha, batch, seq, topk)
    gidx = gidx_row.reshape(batch * topk)
    cand2d = _sc_gather(x2d, gidx, batch * topk)
    return sel_indices, cand2d.reshape(batch, topk, d)


# speedup vs baseline: 1.1053x; 1.0057x over previous
import functools

import jax
import jax.numpy as jnp
import numpy as np
from jax import lax
from jax.experimental import pallas as pl
from jax.experimental.pallas import tpu as pltpu
from jax.experimental.pallas import tpu_sc as plsc

_C0 = np.float32(0.044715)
_C1 = np.float32(0.7978845608028654)


def _gelu(v):
    s3 = jnp.minimum(_C0 * ((v * v) * v), np.float32(3.0e38))
    t = jnp.tanh(_C1 * (v + s3))
    return (np.float32(0.5) * v) * (t + np.float32(1.0))


def _mlp_body(x_ref, w0_ref, b0_ref, w1_ref, b1_ref, wat_ref, alpha_ref):
    h = _gelu(jnp.dot(x_ref[...], w0_ref[...]) + b0_ref[...])
    h2 = _gelu(jnp.dot(h, w1_ref[...]) + b1_ref[...])
    alpha_ref[...] = lax.dot_general(wat_ref[...], h2,
                                     (((1,), (1,)), ((), ())))


def _mlp_scores(x2d, W0, b0, W1, b1, Wa, block_m):
    m, d = x2d.shape
    d0 = W0.shape[1]
    d1 = W1.shape[1]
    grid = (m // block_m,)
    return pl.pallas_call(
        _mlp_body,
        grid=grid,
        in_specs=[
            pl.BlockSpec((block_m, d), lambda i: (i, 0)),
            pl.BlockSpec((d, d0), lambda i: (0, 0)),
            pl.BlockSpec((1, d0), lambda i: (0, 0)),
            pl.BlockSpec((d0, d1), lambda i: (0, 0)),
            pl.BlockSpec((1, d1), lambda i: (0, 0)),
            pl.BlockSpec((1, d1), lambda i: (0, 0)),
        ],
        out_specs=pl.BlockSpec((1, block_m), lambda i: (0, i)),
        out_shape=jax.ShapeDtypeStruct((1, m), jnp.float32),
    )(x2d, W0, b0.reshape(1, d0), W1, b1.reshape(1, d1), Wa.reshape(1, d1))


def _select_body(alpha_ref, sel_ref, gidx_ref, *, batch, seq, topk, chunk):
    j_row = lax.broadcasted_iota(jnp.int32, (1, seq), 1)
    r_row = lax.broadcasted_iota(jnp.int32, (1, topk), 1)
    for b in range(batch):
        v = alpha_ref[0:1, b * seq:(b + 1) * seq]
        m = jnp.max(v)
        e = jnp.exp(v - m)
        s = jnp.sum(e)
        p_row = e / s
        sel_acc = jnp.zeros((1, topk), jnp.int32)
        for c in range(seq // chunk):
            a_col = jnp.reshape(
                alpha_ref[0:1, b * seq + c * chunk:b * seq + (c + 1) * chunk],
                (chunk, 1))
            p_col = jnp.exp(a_col - m) / s
            i_col = (lax.broadcasted_iota(jnp.int32, (chunk, 1), 0)
                     + c * chunk)
            beats = (p_row > p_col) | ((p_row == p_col) & (j_row < i_col))
            rank_c = jnp.sum(beats.astype(jnp.int32), axis=1, keepdims=True)
            hit = rank_c == r_row
            sel_acc = sel_acc + jnp.sum(jnp.where(hit, i_col, 0), axis=0,
                                        keepdims=True)
        sel_ref[b:b + 1, :] = sel_acc
        gidx_ref[0:1, b * topk:(b + 1) * topk] = sel_acc + b * seq


def _select(alpha, batch, seq, topk):
    body = functools.partial(_select_body, batch=batch, seq=seq, topk=topk,
                             chunk=512)
    return pl.pallas_call(
        body,
        out_shape=(
            jax.ShapeDtypeStruct((batch, topk), jnp.int32),
            jax.ShapeDtypeStruct((1, batch * topk), jnp.int32),
        ),
    )(alpha)


def _sc_gather(x2d, gidx, num_rows):
    d = x2d.shape[1]
    info = plsc.get_sparse_core_info()
    nw = info.num_cores * info.num_subcores
    rows_per_w = num_rows // nw
    mesh = plsc.VectorSubcoreMesh(core_axis_name="c", subcore_axis_name="s")

    @functools.partial(
        pl.kernel,
        mesh=mesh,
        out_type=jax.ShapeDtypeStruct((num_rows, d), jnp.float32),
        scratch_types=[
            pltpu.VMEM((rows_per_w,), jnp.int32),
            pltpu.VMEM((rows_per_w, d), jnp.float32),
            pltpu.SemaphoreType.DMA,
        ],
    )
    def gather_kernel(table_hbm, idx_hbm, out_hbm, idx_v, rows_v, sem):
        wid = lax.axis_index("s") * info.num_cores + lax.axis_index("c")
        base = wid * rows_per_w
        pltpu.sync_copy(idx_hbm.at[pl.ds(base, rows_per_w)], idx_v)
        pltpu.async_copy(table_hbm.at[idx_v], rows_v, sem).wait()
        pltpu.sync_copy(rows_v, out_hbm.at[pl.ds(base, rows_per_w)])

    return gather_kernel(x2d, gidx)


def kernel(x, W0, b0, W1, b1, Wa, ba):
    topk = 256
    batch, seq, d = x.shape
    x2d = x.reshape(batch * seq, d)
    alpha = _mlp_scores(x2d, W0, b0, W1, b1, Wa, block_m=1024)
    sel_indices, gidx_row = _select(alpha, batch, seq, topk)
    gidx = gidx_row.reshape(batch * topk)
    cand2d = _sc_gather(x2d, gidx, batch * topk)
    return sel_indices, cand2d.reshape(batch, topk, d)
